# baseline (speedup 1.0000x reference)
import functools

import jax
import jax.numpy as jnp
from jax import lax
from jax.experimental import pallas as pl
from jax.experimental.pallas import tpu as pltpu
from jax.experimental.pallas import tpu_sc as plsc

N = 10000
E = 320000
D = 128
FC = 128
C = 10
B = 64

K = 128
NSUB = 16
RPW = 632
NP = RPW * NSUB
EP = 327680
ER = EP // K
RW1 = ER // 32
RW2 = ER // NSUB
G = 16
R = 1000


def _sc_agg0_body(x_hbm, src_hbm, dst_hbm, zrows_hbm, ones_hbm,
                  acc_out, deg_out,
                  src_c, dst_c, rows_a, rows_b,
                  acc_s, sem_a, sem_b):
    c = lax.axis_index("c")
    s = lax.axis_index("s")
    r0 = s * RPW
    pltpu.sync_copy(zrows_hbm.at[pl.ds(r0, RPW)], acc_s.at[pl.ds(r0, RPW)])
    row0 = (c * NSUB + s) * RW1
    plsc.subcore_barrier()

    def outer(ci, carry):
        rb = row0 + ci * G
        pltpu.sync_copy(src_hbm.at[pl.ds(rb, G)], src_c)
        pltpu.sync_copy(dst_hbm.at[pl.ds(rb, G)], dst_c)

        def inner(g, carry2):
            pltpu.async_copy(x_hbm.at[src_c.at[g]], rows_a, sem_a).wait()
            pltpu.sync_copy(rows_a, acc_s.at[dst_c.at[g]], add=True)
            return carry2

        lax.fori_loop(0, G, inner, 0)
        return carry

    lax.fori_loop(0, RW1 // G, outer, 0)
    plsc.subcore_barrier()
    pltpu.sync_copy(acc_s.at[pl.ds(r0, RPW)], acc_out.at[c, pl.ds(r0, RPW)])
    plsc.subcore_barrier()

    pltpu.sync_copy(zrows_hbm.at[pl.ds(r0, RPW)], acc_s.at[pl.ds(r0, RPW)])
    pltpu.sync_copy(ones_hbm, rows_a)
    plsc.subcore_barrier()

    def douter(ci, carry):
        rb = row0 + ci * G
        pltpu.sync_copy(dst_hbm.at[pl.ds(rb, G)], dst_c)

        def dinner(g, carry2):
            pltpu.sync_copy(rows_a, acc_s.at[dst_c.at[g]], add=True)
            return carry2

        lax.fori_loop(0, G, dinner, 0)
        return carry

    lax.fori_loop(0, RW1 // G, douter, 0)
    plsc.subcore_barrier()
    pltpu.sync_copy(acc_s.at[pl.ds(r0, RPW)], deg_out.at[c, pl.ds(r0, RPW)])


def _sc_agg1_body(m_hbm, v_hbm, src_hbm, dst_hbm, zrows_hbm,
                  agg_out,
                  src_c, dst_c, rows_a, rows_b, acc_s, sem_a, sem_b):
    c = lax.axis_index("c")
    s = lax.axis_index("s")
    r0 = s * RPW
    pltpu.sync_copy(zrows_hbm.at[pl.ds(r0, RPW)], acc_s.at[pl.ds(r0, RPW)])
    row0 = s * RW2
    plsc.subcore_barrier()

    def run(h_hbm):
        def outer(ci, carry):
            rb = row0 + ci * G
            pltpu.sync_copy(src_hbm.at[pl.ds(rb, G)], src_c)
            pltpu.sync_copy(dst_hbm.at[pl.ds(rb, G)], dst_c)

            def inner(g, carry2):
                pltpu.async_copy(h_hbm.at[src_c.at[g]], rows_a, sem_a).wait()
                pltpu.sync_copy(rows_a, acc_s.at[dst_c.at[g]], add=True)
                return carry2

            lax.fori_loop(0, G, inner, 0)
            return carry

        lax.fori_loop(0, RW2 // G, outer, 0)

    @pl.when(c == 0)
    def _():
        run(m_hbm)

    @pl.when(c == 1)
    def _():
        run(v_hbm)

    plsc.subcore_barrier()
    pltpu.sync_copy(acc_s.at[pl.ds(r0, RPW)], agg_out.at[c, pl.ds(r0, RPW)])


@functools.lru_cache(maxsize=None)
def _sc_kernels():
    mesh = plsc.VectorSubcoreMesh(core_axis_name="c", subcore_axis_name="s")
    agg0 = pl.kernel(
        _sc_agg0_body,
        mesh=mesh,
        out_type=[
            jax.ShapeDtypeStruct((2, NP, D), jnp.float32),
            jax.ShapeDtypeStruct((2, NP, D), jnp.float32),
        ],
        scratch_types=[
            pltpu.VMEM((G, K), jnp.int32),
            pltpu.VMEM((G, K), jnp.int32),
            pltpu.VMEM((K, D), jnp.float32),
            pltpu.VMEM((K, D), jnp.float32),
            pltpu.VMEM_SHARED((NP, D), jnp.float32),
            pltpu.SemaphoreType.DMA,
            pltpu.SemaphoreType.DMA,
        ],
    )
    agg1 = pl.kernel(
        _sc_agg1_body,
        mesh=mesh,
        out_type=jax.ShapeDtypeStruct((2, NP, D), jnp.float32),
        scratch_types=[
            pltpu.VMEM((G, K), jnp.int32),
            pltpu.VMEM((G, K), jnp.int32),
            pltpu.VMEM((K, D), jnp.float32),
            pltpu.VMEM((K, D), jnp.float32),
            pltpu.VMEM_SHARED((NP, D), jnp.float32),
            pltpu.SemaphoreType.DMA,
            pltpu.SemaphoreType.DMA,
        ],
    )
    return agg0, agg1


def _stage_a_body(x_ref, accp_ref, degp_ref,
                  mWl0_ref, mbl0_ref, mWr0_ref,
                  vWl0_ref, vbl0_ref, vWr0_ref,
                  m_ref, v_ref):
    deg = degp_ref[0, :, 0:1] + degp_ref[1, :, 0:1]
    agg = (accp_ref[0] + accp_ref[1]) / jnp.maximum(deg, 1.0)
    x = x_ref[...]
    m_ref[...] = jnp.maximum(
        jnp.dot(agg, mWl0_ref[...], preferred_element_type=jnp.float32)
        + mbl0_ref[...]
        + jnp.dot(x, mWr0_ref[...], preferred_element_type=jnp.float32), 0.0)
    v_ref[...] = jnp.maximum(
        jnp.dot(agg, vWl0_ref[...], preferred_element_type=jnp.float32)
        + vbl0_ref[...]
        + jnp.dot(x, vWr0_ref[...], preferred_element_type=jnp.float32), 0.0)


def _stage_a(x, accp, degp, mWl0, mbl0, mWr0, vWl0, vbl0, vWr0):
    w_spec = pl.BlockSpec((D, D), lambda i: (0, 0))
    b_spec = pl.BlockSpec((1, D), lambda i: (0, 0))
    return pl.pallas_call(
        _stage_a_body,
        grid=(N // R,),
        in_specs=[
            pl.BlockSpec((R, D), lambda i: (i, 0)),
            pl.BlockSpec((2, R, D), lambda i: (0, i, 0)),
            pl.BlockSpec((2, R, D), lambda i: (0, i, 0)),
            w_spec, b_spec, w_spec,
            w_spec, b_spec, w_spec,
        ],
        out_specs=[
            pl.BlockSpec((R, D), lambda i: (i, 0)),
            pl.BlockSpec((R, D), lambda i: (i, 0)),
        ],
        out_shape=[
            jax.ShapeDtypeStruct((N, D), jnp.float32),
            jax.ShapeDtypeStruct((N, D), jnp.float32),
        ],
        compiler_params=pltpu.CompilerParams(
            dimension_semantics=("arbitrary",)),
    )(x, accp, degp, mWl0, mbl0.reshape(1, D), mWr0,
      vWl0, vbl0.reshape(1, D), vWr0)


def _stage_b_body(batch_ref, m_ref, v_ref, agg_ref, degp_ref, eps_ref,
                  mWl1_ref, mbl1_ref, mWr1_ref,
                  vWl1_ref, vbl1_ref, vWr1_ref,
                  fc1W_ref, fc1b_ref, fc2W_ref, fc2b_ref,
                  mean_out, lv_out, logp_out,
                  pooled_scr, cnt_scr):
    i = pl.program_id(0)
    deg = degp_ref[0, :, 0:1] + degp_ref[1, :, 0:1]
    inv_deg = jnp.maximum(deg, 1.0)
    agg_m = agg_ref[0] / inv_deg
    agg_v = agg_ref[1] / inv_deg
    mean2 = jnp.maximum(
        jnp.dot(agg_m, mWl1_ref[...], preferred_element_type=jnp.float32)
        + mbl1_ref[...]
        + jnp.dot(m_ref[...], mWr1_ref[...],
                  preferred_element_type=jnp.float32), 0.0)
    lv2 = jnp.maximum(
        jnp.dot(agg_v, vWl1_ref[...], preferred_element_type=jnp.float32)
        + vbl1_ref[...]
        + jnp.dot(v_ref[...], vWr1_ref[...],
                  preferred_element_type=jnp.float32), 0.0)
    mean_out[...] = mean2
    lv_out[...] = lv2
    z = mean2 + eps_ref[...] * jnp.exp(0.5 * lv2)

    b = batch_ref[0, 0, :]
    onehot = (b[:, None] == lax.broadcasted_iota(jnp.int32, (R, B), 1)
              ).astype(jnp.float32)

    @pl.when(i == 0)
    def _():
        pooled_scr[...] = jnp.zeros((B, D), jnp.float32)
        cnt_scr[...] = jnp.zeros((B, D), jnp.float32)

    pooled_scr[...] += lax.dot_general(
        onehot, z, (((0,), (0,)), ((), ())),
        preferred_element_type=jnp.float32)
    cnt_scr[...] += jnp.broadcast_to(jnp.sum(onehot, axis=0)[:, None], (B, D))

    @pl.when(i == (N // R) - 1)
    def _():
        pooled = pooled_scr[...] / jnp.maximum(cnt_scr[...], 1.0)
        a = jnp.maximum(
            jnp.dot(pooled, fc1W_ref[...],
                    preferred_element_type=jnp.float32) + fc1b_ref[...], 0.0)
        logits = jnp.dot(a, fc2W_ref[...],
                         preferred_element_type=jnp.float32) + fc2b_ref[...]
        logp_out[...] = jax.nn.log_softmax(logits, axis=1)


def _stage_b(batch3, m, v, agg1, degp, eps,
             mWl1, mbl1, mWr1, vWl1, vbl1, vWr1, fc1W, fc1b, fc2W, fc2b):
    w_spec = pl.BlockSpec((D, D), lambda i: (0, 0))
    b_spec = pl.BlockSpec((1, D), lambda i: (0, 0))
    row_spec = pl.BlockSpec((R, D), lambda i: (i, 0))
    return pl.pallas_call(
        _stage_b_body,
        grid=(N // R,),
        in_specs=[
            pl.BlockSpec((1, 1, R), lambda i: (i, 0, 0)),
            row_spec, row_spec,
            pl.BlockSpec((2, R, D), lambda i: (0, i, 0)),
            pl.BlockSpec((2, R, D), lambda i: (0, i, 0)),
            row_spec,
            w_spec, b_spec, w_spec,
            w_spec, b_spec, w_spec,
            pl.BlockSpec((D, FC), lambda i: (0, 0)),
            pl.BlockSpec((1, FC), lambda i: (0, 0)),
            pl.BlockSpec((FC, C), lambda i: (0, 0)),
            pl.BlockSpec((1, C), lambda i: (0, 0)),
        ],
        out_specs=[
            row_spec, row_spec,
            pl.BlockSpec((B, C), lambda i: (0, 0)),
        ],
        out_shape=[
            jax.ShapeDtypeStruct((N, D), jnp.float32),
            jax.ShapeDtypeStruct((N, D), jnp.float32),
            jax.ShapeDtypeStruct((B, C), jnp.float32),
        ],
        scratch_shapes=[
            pltpu.VMEM((B, D), jnp.float32),
            pltpu.VMEM((B, D), jnp.float32),
        ],
        compiler_params=pltpu.CompilerParams(
            dimension_semantics=("arbitrary",)),
    )(batch3, m, v, agg1, degp, eps,
      mWl1, mbl1.reshape(1, D), mWr1, vWl1, vbl1.reshape(1, D), vWr1,
      fc1W, fc1b.reshape(1, FC), fc2W, fc2b.reshape(1, C))


def kernel(x, edge_index, batch,
           mWl0, mbl0, mWr0, mWl1, mbl1, mWr1,
           vWl0, vbl0, vWr0, vWl1, vbl1, vWr1,
           fc1W, fc1b, fc2W, fc2b):
    pad = EP - E
    src2 = jnp.concatenate(
        [edge_index[0], jnp.zeros((pad,), jnp.int32)]).reshape(ER, K)
    trash = N + jnp.arange(pad, dtype=jnp.int32) % (NP - N)
    dst2 = jnp.concatenate([edge_index[1], trash]).reshape(ER, K)
    zrows = jnp.zeros((NP, D), jnp.float32)
    ones_k = jnp.ones((K, D), jnp.float32)

    sc_agg0, sc_agg1 = _sc_kernels()
    accp, degp = sc_agg0(x, src2, dst2, zrows, ones_k)
    accp = accp[:, :N]
    degp = degp[:, :N]
    m, v = _stage_a(x, accp, degp, mWl0, mbl0, mWr0, vWl0, vbl0, vWr0)
    agg1 = sc_agg1(m, v, src2, dst2, zrows)[:, :N]

    eps = jax.random.normal(jax.random.key(42), (N, D), dtype=jnp.float32)
    batch3 = batch.reshape(N // R, 1, R)
    mean2, lv2, logp = _stage_b(
        batch3, m, v, agg1, degp, eps,
        mWl1, mbl1, mWr1, vWl1, vbl1, vWr1, fc1W, fc1b, fc2W, fc2b)
    return (logp, mean2, lv2)


# speedup vs baseline: 1.7239x; 1.7239x over previous
import functools

import jax
import jax.numpy as jnp
from jax import lax
from jax.experimental import pallas as pl
from jax.experimental.pallas import tpu as pltpu
from jax.experimental.pallas import tpu_sc as plsc

N = 10000
E = 320000
D = 128
FC = 128
C = 10
B = 64

K = 80
NSUB = 16
RPW = 632
NP = RPW * NSUB
R = 1000


def _sc_agg0_body(x_hbm, src_hbm, dst_hbm, zrows_hbm, ones_hbm,
                  acc_out, deg_out,
                  src_a, dst_a, src_b, dst_b, rows_a, rows_b, ones_v,
                  acc_s, sem_a, sem_b):
    c = lax.axis_index("c")
    s = lax.axis_index("s")
    r0 = s * RPW
    pltpu.sync_copy(zrows_hbm.at[pl.ds(r0, RPW)], acc_s.at[pl.ds(r0, RPW)])
    pltpu.sync_copy(ones_hbm, ones_v)
    plsc.subcore_barrier()

    epw = E // 32
    base = c * (E // 2) + s * epw
    steps = epw // K

    def step2(i, carry):
        t0 = base + 2 * i * K
        pltpu.sync_copy(src_hbm.at[pl.ds(t0, K)], src_a)
        pltpu.sync_copy(dst_hbm.at[pl.ds(t0, K)], dst_a)
        ha = pltpu.async_copy(x_hbm.at[src_a], rows_a, sem_a)
        pltpu.sync_copy(src_hbm.at[pl.ds(t0 + K, K)], src_b)
        pltpu.sync_copy(dst_hbm.at[pl.ds(t0 + K, K)], dst_b)
        hb = pltpu.async_copy(x_hbm.at[src_b], rows_b, sem_b)
        ha.wait()
        pltpu.sync_copy(rows_a, acc_s.at[dst_a], add=True)
        hb.wait()
        pltpu.sync_copy(rows_b, acc_s.at[dst_b], add=True)
        return carry

    lax.fori_loop(0, steps // 2, step2, 0)
    t_last = base + (steps - 1) * K
    pltpu.sync_copy(src_hbm.at[pl.ds(t_last, K)], src_a)
    pltpu.sync_copy(dst_hbm.at[pl.ds(t_last, K)], dst_a)
    pltpu.async_copy(x_hbm.at[src_a], rows_a, sem_a).wait()
    pltpu.sync_copy(rows_a, acc_s.at[dst_a], add=True)

    plsc.subcore_barrier()
    pltpu.sync_copy(acc_s.at[pl.ds(r0, RPW)], acc_out.at[c, pl.ds(r0, RPW)])
    plsc.subcore_barrier()

    pltpu.sync_copy(zrows_hbm.at[pl.ds(r0, RPW)], acc_s.at[pl.ds(r0, RPW)])
    plsc.subcore_barrier()

    def dstep(t, carry):
        off = base + t * K
        pltpu.sync_copy(dst_hbm.at[pl.ds(off, K)], dst_a)
        pltpu.sync_copy(ones_v, acc_s.at[dst_a], add=True)
        return carry

    lax.fori_loop(0, steps, dstep, 0)
    plsc.subcore_barrier()
    pltpu.sync_copy(acc_s.at[pl.ds(r0, RPW)], deg_out.at[c, pl.ds(r0, RPW)])


def _sc_agg1_body(m_hbm, v_hbm, src_hbm, dst_hbm, zrows_hbm,
                  agg_out,
                  src_a, dst_a, src_b, dst_b, rows_a, rows_b,
                  acc_s, sem_a, sem_b):
    c = lax.axis_index("c")
    s = lax.axis_index("s")
    r0 = s * RPW
    pltpu.sync_copy(zrows_hbm.at[pl.ds(r0, RPW)], acc_s.at[pl.ds(r0, RPW)])
    plsc.subcore_barrier()

    epw = E // NSUB
    base = s * epw
    steps = epw // K

    def run(h_hbm):
        def step2(i, carry):
            t0 = base + 2 * i * K
            pltpu.sync_copy(src_hbm.at[pl.ds(t0, K)], src_a)
            pltpu.sync_copy(dst_hbm.at[pl.ds(t0, K)], dst_a)
            ha = pltpu.async_copy(h_hbm.at[src_a], rows_a, sem_a)
            pltpu.sync_copy(src_hbm.at[pl.ds(t0 + K, K)], src_b)
            pltpu.sync_copy(dst_hbm.at[pl.ds(t0 + K, K)], dst_b)
            hb = pltpu.async_copy(h_hbm.at[src_b], rows_b, sem_b)
            ha.wait()
            pltpu.sync_copy(rows_a, acc_s.at[dst_a], add=True)
            hb.wait()
            pltpu.sync_copy(rows_b, acc_s.at[dst_b], add=True)
            return carry
        lax.fori_loop(0, steps // 2, step2, 0)

    @pl.when(c == 0)
    def _():
        run(m_hbm)

    @pl.when(c == 1)
    def _():
        run(v_hbm)

    plsc.subcore_barrier()
    pltpu.sync_copy(acc_s.at[pl.ds(r0, RPW)], agg_out.at[c, pl.ds(r0, RPW)])


@functools.lru_cache(maxsize=None)
def _sc_kernels():
    mesh = plsc.VectorSubcoreMesh(core_axis_name="c", subcore_axis_name="s")
    agg0 = pl.kernel(
        _sc_agg0_body,
        mesh=mesh,
        out_type=[
            jax.ShapeDtypeStruct((2, NP, D), jnp.float32),
            jax.ShapeDtypeStruct((2, NP, D), jnp.float32),
        ],
        scratch_types=[
            pltpu.VMEM((K,), jnp.int32),
            pltpu.VMEM((K,), jnp.int32),
            pltpu.VMEM((K,), jnp.int32),
            pltpu.VMEM((K,), jnp.int32),
            pltpu.VMEM((K, D), jnp.float32),
            pltpu.VMEM((K, D), jnp.float32),
            pltpu.VMEM((K, D), jnp.float32),
            pltpu.VMEM_SHARED((NP, D), jnp.float32),
            pltpu.SemaphoreType.DMA,
            pltpu.SemaphoreType.DMA,
        ],
    )
    agg1 = pl.kernel(
        _sc_agg1_body,
        mesh=mesh,
        out_type=jax.ShapeDtypeStruct((2, NP, D), jnp.float32),
        scratch_types=[
            pltpu.VMEM((K,), jnp.int32),
            pltpu.VMEM((K,), jnp.int32),
            pltpu.VMEM((K,), jnp.int32),
            pltpu.VMEM((K,), jnp.int32),
            pltpu.VMEM((K, D), jnp.float32),
            pltpu.VMEM((K, D), jnp.float32),
            pltpu.VMEM_SHARED((NP, D), jnp.float32),
            pltpu.SemaphoreType.DMA,
            pltpu.SemaphoreType.DMA,
        ],
    )
    return agg0, agg1


def _stage_a_body(x_ref, accp_ref, degp_ref,
                  mWl0_ref, mbl0_ref, mWr0_ref,
                  vWl0_ref, vbl0_ref, vWr0_ref,
                  m_ref, v_ref):
    deg = degp_ref[0, :, 0:1] + degp_ref[1, :, 0:1]
    agg = (accp_ref[0] + accp_ref[1]) / jnp.maximum(deg, 1.0)
    x = x_ref[...]
    m_ref[...] = jnp.maximum(
        jnp.dot(agg, mWl0_ref[...], preferred_element_type=jnp.float32)
        + mbl0_ref[...]
        + jnp.dot(x, mWr0_ref[...], preferred_element_type=jnp.float32), 0.0)
    v_ref[...] = jnp.maximum(
        jnp.dot(agg, vWl0_ref[...], preferred_element_type=jnp.float32)
        + vbl0_ref[...]
        + jnp.dot(x, vWr0_ref[...], preferred_element_type=jnp.float32), 0.0)


def _stage_a(x, accp, degp, mWl0, mbl0, mWr0, vWl0, vbl0, vWr0):
    w_spec = pl.BlockSpec((D, D), lambda i: (0, 0))
    b_spec = pl.BlockSpec((1, D), lambda i: (0, 0))
    return pl.pallas_call(
        _stage_a_body,
        grid=(N // R,),
        in_specs=[
            pl.BlockSpec((R, D), lambda i: (i, 0)),
            pl.BlockSpec((2, R, D), lambda i: (0, i, 0)),
            pl.BlockSpec((2, R, D), lambda i: (0, i, 0)),
            w_spec, b_spec, w_spec,
            w_spec, b_spec, w_spec,
        ],
        out_specs=[
            pl.BlockSpec((R, D), lambda i: (i, 0)),
            pl.BlockSpec((R, D), lambda i: (i, 0)),
        ],
        out_shape=[
            jax.ShapeDtypeStruct((N, D), jnp.float32),
            jax.ShapeDtypeStruct((N, D), jnp.float32),
        ],
        compiler_params=pltpu.CompilerParams(
            dimension_semantics=("arbitrary",)),
    )(x, accp, degp, mWl0, mbl0.reshape(1, D), mWr0,
      vWl0, vbl0.reshape(1, D), vWr0)


def _stage_b_body(batch_ref, m_ref, v_ref, agg_ref, degp_ref, eps_ref,
                  mWl1_ref, mbl1_ref, mWr1_ref,
                  vWl1_ref, vbl1_ref, vWr1_ref,
                  fc1W_ref, fc1b_ref, fc2W_ref, fc2b_ref,
                  mean_out, lv_out, logp_out,
                  pooled_scr, cnt_scr):
    i = pl.program_id(0)
    deg = degp_ref[0, :, 0:1] + degp_ref[1, :, 0:1]
    inv_deg = jnp.maximum(deg, 1.0)
    agg_m = agg_ref[0] / inv_deg
    agg_v = agg_ref[1] / inv_deg
    mean2 = jnp.maximum(
        jnp.dot(agg_m, mWl1_ref[...], preferred_element_type=jnp.float32)
        + mbl1_ref[...]
        + jnp.dot(m_ref[...], mWr1_ref[...],
                  preferred_element_type=jnp.float32), 0.0)
    lv2 = jnp.maximum(
        jnp.dot(agg_v, vWl1_ref[...], preferred_element_type=jnp.float32)
        + vbl1_ref[...]
        + jnp.dot(v_ref[...], vWr1_ref[...],
                  preferred_element_type=jnp.float32), 0.0)
    mean_out[...] = mean2
    lv_out[...] = lv2
    z = mean2 + eps_ref[...] * jnp.exp(0.5 * lv2)

    b = batch_ref[0, 0, :]
    onehot = (b[:, None] == lax.broadcasted_iota(jnp.int32, (R, B), 1)
              ).astype(jnp.float32)

    @pl.when(i == 0)
    def _():
        pooled_scr[...] = jnp.zeros((B, D), jnp.float32)
        cnt_scr[...] = jnp.zeros((B, D), jnp.float32)

    pooled_scr[...] += lax.dot_general(
        onehot, z, (((0,), (0,)), ((), ())),
        preferred_element_type=jnp.float32)
    cnt_scr[...] += jnp.broadcast_to(jnp.sum(onehot, axis=0)[:, None], (B, D))

    @pl.when(i == (N // R) - 1)
    def _():
        pooled = pooled_scr[...] / jnp.maximum(cnt_scr[...], 1.0)
        a = jnp.maximum(
            jnp.dot(pooled, fc1W_ref[...],
                    preferred_element_type=jnp.float32) + fc1b_ref[...], 0.0)
        logits = jnp.dot(a, fc2W_ref[...],
                         preferred_element_type=jnp.float32) + fc2b_ref[...]
        logp_out[...] = jax.nn.log_softmax(logits, axis=1)


def _stage_b(batch3, m, v, agg1, degp, eps,
             mWl1, mbl1, mWr1, vWl1, vbl1, vWr1, fc1W, fc1b, fc2W, fc2b):
    w_spec = pl.BlockSpec((D, D), lambda i: (0, 0))
    b_spec = pl.BlockSpec((1, D), lambda i: (0, 0))
    row_spec = pl.BlockSpec((R, D), lambda i: (i, 0))
    return pl.pallas_call(
        _stage_b_body,
        grid=(N // R,),
        in_specs=[
            pl.BlockSpec((1, 1, R), lambda i: (i, 0, 0)),
            row_spec, row_spec,
            pl.BlockSpec((2, R, D), lambda i: (0, i, 0)),
            pl.BlockSpec((2, R, D), lambda i: (0, i, 0)),
            row_spec,
            w_spec, b_spec, w_spec,
            w_spec, b_spec, w_spec,
            pl.BlockSpec((D, FC), lambda i: (0, 0)),
            pl.BlockSpec((1, FC), lambda i: (0, 0)),
            pl.BlockSpec((FC, C), lambda i: (0, 0)),
            pl.BlockSpec((1, C), lambda i: (0, 0)),
        ],
        out_specs=[
            row_spec, row_spec,
            pl.BlockSpec((B, C), lambda i: (0, 0)),
        ],
        out_shape=[
            jax.ShapeDtypeStruct((N, D), jnp.float32),
            jax.ShapeDtypeStruct((N, D), jnp.float32),
            jax.ShapeDtypeStruct((B, C), jnp.float32),
        ],
        scratch_shapes=[
            pltpu.VMEM((B, D), jnp.float32),
            pltpu.VMEM((B, D), jnp.float32),
        ],
        compiler_params=pltpu.CompilerParams(
            dimension_semantics=("arbitrary",)),
    )(batch3, m, v, agg1, degp, eps,
      mWl1, mbl1.reshape(1, D), mWr1, vWl1, vbl1.reshape(1, D), vWr1,
      fc1W, fc1b.reshape(1, FC), fc2W, fc2b.reshape(1, C))


def kernel(x, edge_index, batch,
           mWl0, mbl0, mWr0, mWl1, mbl1, mWr1,
           vWl0, vbl0, vWr0, vWl1, vbl1, vWr1,
           fc1W, fc1b, fc2W, fc2b):
    src = edge_index[0]
    dst = edge_index[1]
    zrows = jnp.zeros((NP, D), jnp.float32)
    ones_k = jnp.ones((K, D), jnp.float32)

    sc_agg0, sc_agg1 = _sc_kernels()
    accp, degp = sc_agg0(x, src, dst, zrows, ones_k)
    accp = accp[:, :N]
    degp = degp[:, :N]
    m, v = _stage_a(x, accp, degp, mWl0, mbl0, mWr0, vWl0, vbl0, vWr0)
    agg1 = sc_agg1(m, v, src, dst, zrows)[:, :N]

    eps = jax.random.normal(jax.random.key(42), (N, D), dtype=jnp.float32)
    batch3 = batch.reshape(N // R, 1, R)
    mean2, lv2, logp = _stage_b(
        batch3, m, v, agg1, degp, eps,
        mWl1, mbl1, mWr1, vWl1, vbl1, vWr1, fc1W, fc1b, fc2W, fc2b)
    return (logp, mean2, lv2)


# speedup vs baseline: 1.8202x; 1.0559x over previous
import functools

import jax
import jax.numpy as jnp
from jax import lax
from jax.experimental import pallas as pl
from jax.experimental.pallas import tpu as pltpu
from jax.experimental.pallas import tpu_sc as plsc

N = 10000
E = 320000
D = 128
FC = 128
C = 10
B = 64

K = 80
NSUB = 16
RPW = 632
NP = RPW * NSUB
R = 1000


def _sc_agg0_body(x_hbm, src_hbm, dst_hbm, zrows_hbm, ones_hbm,
                  acc_out, deg_out,
                  src_a, dst_a, src_b, dst_b, rows_a, rows_b, ones_v,
                  acc_s, sem_a, sem_b):
    c = lax.axis_index("c")
    s = lax.axis_index("s")
    r0 = s * RPW
    pltpu.sync_copy(zrows_hbm.at[pl.ds(r0, RPW)], acc_s.at[pl.ds(r0, RPW)])
    pltpu.sync_copy(ones_hbm, ones_v)
    plsc.subcore_barrier()

    epw = E // 32
    base = c * (E // 2) + s * epw
    steps = epw // K

    def step2(i, carry):
        t0 = base + 2 * i * K
        pltpu.sync_copy(src_hbm.at[pl.ds(t0, K)], src_a)
        pltpu.sync_copy(dst_hbm.at[pl.ds(t0, K)], dst_a)
        ha = pltpu.async_copy(x_hbm.at[src_a], rows_a, sem_a)
        pltpu.sync_copy(src_hbm.at[pl.ds(t0 + K, K)], src_b)
        pltpu.sync_copy(dst_hbm.at[pl.ds(t0 + K, K)], dst_b)
        hb = pltpu.async_copy(x_hbm.at[src_b], rows_b, sem_b)
        ha.wait()
        pltpu.sync_copy(rows_a, acc_s.at[dst_a], add=True)
        hb.wait()
        pltpu.sync_copy(rows_b, acc_s.at[dst_b], add=True)
        return carry

    lax.fori_loop(0, steps // 2, step2, 0)
    t_last = base + (steps - 1) * K
    pltpu.sync_copy(src_hbm.at[pl.ds(t_last, K)], src_a)
    pltpu.sync_copy(dst_hbm.at[pl.ds(t_last, K)], dst_a)
    pltpu.async_copy(x_hbm.at[src_a], rows_a, sem_a).wait()
    pltpu.sync_copy(rows_a, acc_s.at[dst_a], add=True)

    plsc.subcore_barrier()
    pltpu.sync_copy(acc_s.at[pl.ds(r0, RPW)], acc_out.at[c, pl.ds(r0, RPW)])
    plsc.subcore_barrier()

    pltpu.sync_copy(zrows_hbm.at[pl.ds(r0, RPW)], acc_s.at[pl.ds(r0, RPW)])
    plsc.subcore_barrier()

    def dstep2(i, carry):
        t0 = base + 2 * i * K
        ia = pltpu.async_copy(dst_hbm.at[pl.ds(t0, K)], dst_a, sem_a)
        ib = pltpu.async_copy(dst_hbm.at[pl.ds(t0 + K, K)], dst_b, sem_b)
        ia.wait()
        pltpu.sync_copy(ones_v, acc_s.at[dst_a], add=True)
        ib.wait()
        pltpu.sync_copy(ones_v, acc_s.at[dst_b], add=True)
        return carry

    lax.fori_loop(0, steps // 2, dstep2, 0)
    pltpu.sync_copy(dst_hbm.at[pl.ds(base + (steps - 1) * K, K)], dst_a)
    pltpu.sync_copy(ones_v, acc_s.at[dst_a], add=True)
    plsc.subcore_barrier()
    pltpu.sync_copy(acc_s.at[pl.ds(r0, RPW)], deg_out.at[c, pl.ds(r0, RPW)])


def _sc_agg1_body(m_hbm, v_hbm, src_hbm, dst_hbm, zrows_hbm,
                  agg_out,
                  src_a, dst_a, src_b, dst_b, rows_a, rows_b,
                  acc_s, sem_a, sem_b):
    c = lax.axis_index("c")
    s = lax.axis_index("s")
    r0 = s * RPW
    pltpu.sync_copy(zrows_hbm.at[pl.ds(r0, RPW)], acc_s.at[pl.ds(r0, RPW)])
    plsc.subcore_barrier()

    epw = E // NSUB
    base = s * epw
    steps = epw // K

    def run(h_hbm):
        def step2(i, carry):
            t0 = base + 2 * i * K
            pltpu.sync_copy(src_hbm.at[pl.ds(t0, K)], src_a)
            pltpu.sync_copy(dst_hbm.at[pl.ds(t0, K)], dst_a)
            ha = pltpu.async_copy(h_hbm.at[src_a], rows_a, sem_a)
            pltpu.sync_copy(src_hbm.at[pl.ds(t0 + K, K)], src_b)
            pltpu.sync_copy(dst_hbm.at[pl.ds(t0 + K, K)], dst_b)
            hb = pltpu.async_copy(h_hbm.at[src_b], rows_b, sem_b)
            ha.wait()
            pltpu.sync_copy(rows_a, acc_s.at[dst_a], add=True)
            hb.wait()
            pltpu.sync_copy(rows_b, acc_s.at[dst_b], add=True)
            return carry
        lax.fori_loop(0, steps // 2, step2, 0)

    @pl.when(c == 0)
    def _():
        run(m_hbm)

    @pl.when(c == 1)
    def _():
        run(v_hbm)

    plsc.subcore_barrier()
    pltpu.sync_copy(acc_s.at[pl.ds(r0, RPW)], agg_out.at[c, pl.ds(r0, RPW)])


@functools.lru_cache(maxsize=None)
def _sc_kernels():
    mesh = plsc.VectorSubcoreMesh(core_axis_name="c", subcore_axis_name="s")
    agg0 = pl.kernel(
        _sc_agg0_body,
        mesh=mesh,
        out_type=[
            jax.ShapeDtypeStruct((2, NP, D), jnp.float32),
            jax.ShapeDtypeStruct((2, NP, D), jnp.float32),
        ],
        scratch_types=[
            pltpu.VMEM((K,), jnp.int32),
            pltpu.VMEM((K,), jnp.int32),
            pltpu.VMEM((K,), jnp.int32),
            pltpu.VMEM((K,), jnp.int32),
            pltpu.VMEM((K, D), jnp.float32),
            pltpu.VMEM((K, D), jnp.float32),
            pltpu.VMEM((K, D), jnp.float32),
            pltpu.VMEM_SHARED((NP, D), jnp.float32),
            pltpu.SemaphoreType.DMA,
            pltpu.SemaphoreType.DMA,
        ],
    )
    agg1 = pl.kernel(
        _sc_agg1_body,
        mesh=mesh,
        out_type=jax.ShapeDtypeStruct((2, NP, D), jnp.float32),
        scratch_types=[
            pltpu.VMEM((K,), jnp.int32),
            pltpu.VMEM((K,), jnp.int32),
            pltpu.VMEM((K,), jnp.int32),
            pltpu.VMEM((K,), jnp.int32),
            pltpu.VMEM((K, D), jnp.float32),
            pltpu.VMEM((K, D), jnp.float32),
            pltpu.VMEM_SHARED((NP, D), jnp.float32),
            pltpu.SemaphoreType.DMA,
            pltpu.SemaphoreType.DMA,
        ],
    )
    return agg0, agg1


def _stage_a_body(x_ref, accp_ref, degp_ref,
                  mWl0_ref, mbl0_ref, mWr0_ref,
                  vWl0_ref, vbl0_ref, vWr0_ref,
                  m_ref, v_ref):
    deg = degp_ref[0, :, 0:1] + degp_ref[1, :, 0:1]
    agg = (accp_ref[0] + accp_ref[1]) / jnp.maximum(deg, 1.0)
    x = x_ref[...]
    m_ref[...] = jnp.maximum(
        jnp.dot(agg, mWl0_ref[...], preferred_element_type=jnp.float32)
        + mbl0_ref[...]
        + jnp.dot(x, mWr0_ref[...], preferred_element_type=jnp.float32), 0.0)
    v_ref[...] = jnp.maximum(
        jnp.dot(agg, vWl0_ref[...], preferred_element_type=jnp.float32)
        + vbl0_ref[...]
        + jnp.dot(x, vWr0_ref[...], preferred_element_type=jnp.float32), 0.0)


def _stage_a(x, accp, degp, mWl0, mbl0, mWr0, vWl0, vbl0, vWr0):
    w_spec = pl.BlockSpec((D, D), lambda i: (0, 0))
    b_spec = pl.BlockSpec((1, D), lambda i: (0, 0))
    return pl.pallas_call(
        _stage_a_body,
        grid=(N // R,),
        in_specs=[
            pl.BlockSpec((R, D), lambda i: (i, 0)),
            pl.BlockSpec((2, R, D), lambda i: (0, i, 0)),
            pl.BlockSpec((2, R, D), lambda i: (0, i, 0)),
            w_spec, b_spec, w_spec,
            w_spec, b_spec, w_spec,
        ],
        out_specs=[
            pl.BlockSpec((R, D), lambda i: (i, 0)),
            pl.BlockSpec((R, D), lambda i: (i, 0)),
        ],
        out_shape=[
            jax.ShapeDtypeStruct((N, D), jnp.float32),
            jax.ShapeDtypeStruct((N, D), jnp.float32),
        ],
        compiler_params=pltpu.CompilerParams(
            dimension_semantics=("arbitrary",)),
    )(x, accp, degp, mWl0, mbl0.reshape(1, D), mWr0,
      vWl0, vbl0.reshape(1, D), vWr0)


def _stage_b_body(batch_ref, m_ref, v_ref, agg_ref, degp_ref, eps_ref,
                  mWl1_ref, mbl1_ref, mWr1_ref,
                  vWl1_ref, vbl1_ref, vWr1_ref,
                  fc1W_ref, fc1b_ref, fc2W_ref, fc2b_ref,
                  mean_out, lv_out, logp_out,
                  pooled_scr, cnt_scr):
    i = pl.program_id(0)
    deg = degp_ref[0, :, 0:1] + degp_ref[1, :, 0:1]
    inv_deg = jnp.maximum(deg, 1.0)
    agg_m = agg_ref[0] / inv_deg
    agg_v = agg_ref[1] / inv_deg
    mean2 = jnp.maximum(
        jnp.dot(agg_m, mWl1_ref[...], preferred_element_type=jnp.float32)
        + mbl1_ref[...]
        + jnp.dot(m_ref[...], mWr1_ref[...],
                  preferred_element_type=jnp.float32), 0.0)
    lv2 = jnp.maximum(
        jnp.dot(agg_v, vWl1_ref[...], preferred_element_type=jnp.float32)
        + vbl1_ref[...]
        + jnp.dot(v_ref[...], vWr1_ref[...],
                  preferred_element_type=jnp.float32), 0.0)
    mean_out[...] = mean2
    lv_out[...] = lv2
    z = mean2 + eps_ref[...] * jnp.exp(0.5 * lv2)

    b = batch_ref[0, 0, :]
    onehot = (b[:, None] == lax.broadcasted_iota(jnp.int32, (R, B), 1)
              ).astype(jnp.float32)

    @pl.when(i == 0)
    def _():
        pooled_scr[...] = jnp.zeros((B, D), jnp.float32)
        cnt_scr[...] = jnp.zeros((B, D), jnp.float32)

    pooled_scr[...] += lax.dot_general(
        onehot, z, (((0,), (0,)), ((), ())),
        preferred_element_type=jnp.float32)
    cnt_scr[...] += jnp.broadcast_to(jnp.sum(onehot, axis=0)[:, None], (B, D))

    @pl.when(i == (N // R) - 1)
    def _():
        pooled = pooled_scr[...] / jnp.maximum(cnt_scr[...], 1.0)
        a = jnp.maximum(
            jnp.dot(pooled, fc1W_ref[...],
                    preferred_element_type=jnp.float32) + fc1b_ref[...], 0.0)
        logits = jnp.dot(a, fc2W_ref[...],
                         preferred_element_type=jnp.float32) + fc2b_ref[...]
        logp_out[...] = jax.nn.log_softmax(logits, axis=1)


def _stage_b(batch3, m, v, agg1, degp, eps,
             mWl1, mbl1, mWr1, vWl1, vbl1, vWr1, fc1W, fc1b, fc2W, fc2b):
    w_spec = pl.BlockSpec((D, D), lambda i: (0, 0))
    b_spec = pl.BlockSpec((1, D), lambda i: (0, 0))
    row_spec = pl.BlockSpec((R, D), lambda i: (i, 0))
    return pl.pallas_call(
        _stage_b_body,
        grid=(N // R,),
        in_specs=[
            pl.BlockSpec((1, 1, R), lambda i: (i, 0, 0)),
            row_spec, row_spec,
            pl.BlockSpec((2, R, D), lambda i: (0, i, 0)),
            pl.BlockSpec((2, R, D), lambda i: (0, i, 0)),
            row_spec,
            w_spec, b_spec, w_spec,
            w_spec, b_spec, w_spec,
            pl.BlockSpec((D, FC), lambda i: (0, 0)),
            pl.BlockSpec((1, FC), lambda i: (0, 0)),
            pl.BlockSpec((FC, C), lambda i: (0, 0)),
            pl.BlockSpec((1, C), lambda i: (0, 0)),
        ],
        out_specs=[
            row_spec, row_spec,
            pl.BlockSpec((B, C), lambda i: (0, 0)),
        ],
        out_shape=[
            jax.ShapeDtypeStruct((N, D), jnp.float32),
            jax.ShapeDtypeStruct((N, D), jnp.float32),
            jax.ShapeDtypeStruct((B, C), jnp.float32),
        ],
        scratch_shapes=[
            pltpu.VMEM((B, D), jnp.float32),
            pltpu.VMEM((B, D), jnp.float32),
        ],
        compiler_params=pltpu.CompilerParams(
            dimension_semantics=("arbitrary",)),
    )(batch3, m, v, agg1, degp, eps,
      mWl1, mbl1.reshape(1, D), mWr1, vWl1, vbl1.reshape(1, D), vWr1,
      fc1W, fc1b.reshape(1, FC), fc2W, fc2b.reshape(1, C))


def kernel(x, edge_index, batch,
           mWl0, mbl0, mWr0, mWl1, mbl1, mWr1,
           vWl0, vbl0, vWr0, vWl1, vbl1, vWr1,
           fc1W, fc1b, fc2W, fc2b):
    src = edge_index[0]
    dst = edge_index[1]
    zrows = jnp.zeros((NP, D), jnp.float32)
    ones_k = jnp.ones((K, D), jnp.float32)

    sc_agg0, sc_agg1 = _sc_kernels()
    accp, degp = sc_agg0(x, src, dst, zrows, ones_k)
    m, v = _stage_a(x, accp, degp, mWl0, mbl0, mWr0, vWl0, vbl0, vWr0)
    agg1 = sc_agg1(m, v, src, dst, zrows)

    eps = jax.random.normal(jax.random.key(42), (N, D), dtype=jnp.float32)
    batch3 = batch.reshape(N // R, 1, R)
    mean2, lv2, logp = _stage_b(
        batch3, m, v, agg1, degp, eps,
        mWl1, mbl1, mWr1, vWl1, vbl1, vWr1, fc1W, fc1b, fc2W, fc2b)
    return (logp, mean2, lv2)
